# Initial kernel scaffold; baseline (speedup 1.0000x reference)
import jax
import jax.numpy as jnp
from jax.experimental import pallas as pl


def kernel(x, edge_index, edge_weights, feature_mask, W0, b0, W1, b1):
    raise NotImplementedError("write your pallas kernel here")


# speedup vs baseline: 9.9272x; 9.9272x over previous
import functools

import jax
import jax.numpy as jnp
from jax import lax
from jax.experimental import pallas as pl
from jax.experimental.pallas import tpu as pltpu
from jax.experimental.pallas import tpu_sc as plsc

N = 10000
E = 320000
D = 128
H = D // 2
NP = 10240
NC = 2
NS = 16
RPS = NP // NS
ET = 20480
EP = NS * ET
CHUNKS = ET // 128
WCHUNKS = CHUNKS // 2

_mesh = plsc.VectorSubcoreMesh(core_axis_name="c", subcore_axis_name="s")


@functools.partial(
    pl.kernel,
    out_type=jax.ShapeDtypeStruct((NC, NP), jnp.float32),
    mesh=_mesh,
    compiler_params=pltpu.CompilerParams(use_tc_tiling_on_sc=False),
    scratch_types=[
        pltpu.VMEM((WCHUNKS, 128), jnp.int32),
        pltpu.VMEM((WCHUNKS, 128), jnp.float32),
        pltpu.VMEM((RPS,), jnp.float32),
        pltpu.VMEM_SHARED((NP,), jnp.float32),
    ],
)
def _deg_kernel(dstr, ewr, degp, dstb, ewb, zb, dacc):
    c = lax.axis_index("c")
    s = lax.axis_index("s")
    wid = s * NC + c
    pltpu.sync_copy(dstr.at[pl.ds(wid * WCHUNKS, WCHUNKS)], dstb)
    pltpu.sync_copy(ewr.at[pl.ds(wid * WCHUNKS, WCHUNKS)], ewb)
    zer = jnp.zeros((16,), jnp.float32)

    def zbody(i, _):
        zb[pl.ds(i * 16, 16)] = zer
        return 0

    lax.fori_loop(0, RPS // 16, zbody, 0)
    pltpu.sync_copy(zb, dacc.at[pl.ds(s * RPS, RPS)])
    plsc.subcore_barrier()

    def body(j, _):
        pltpu.sync_copy(ewb.at[j], dacc.at[dstb.at[j]], add=True)
        return 0

    lax.fori_loop(0, WCHUNKS, body, 0)
    plsc.subcore_barrier()
    pltpu.sync_copy(dacc.at[pl.ds(s * RPS, RPS)],
                    degp.at[c, pl.ds(s * RPS, RPS)])


def _dinv_body(degr_ref, dinv_ref, dsq_ref):
    d = degr_ref[0:NP // 128, :] + degr_ref[NP // 128:2 * (NP // 128), :]
    pos = d > 0
    dinv_ref[...] = jnp.where(pos, lax.rsqrt(d), 0.0)
    dsq_ref[...] = jnp.where(pos, jnp.sqrt(d), 0.0)


def _dinv_kernel(degr):
    return pl.pallas_call(
        _dinv_body,
        out_shape=(jax.ShapeDtypeStruct((NP // 128, 128), jnp.float32),
                   jax.ShapeDtypeStruct((NP // 128, 128), jnp.float32)),
    )(degr)


@functools.partial(
    pl.kernel,
    out_type=jax.ShapeDtypeStruct((2 * NP, H), jnp.float32),
    mesh=_mesh,
    compiler_params=pltpu.CompilerParams(use_tc_tiling_on_sc=False),
    scratch_types=[
        pltpu.VMEM((CHUNKS, 128), jnp.int32),
        pltpu.VMEM((CHUNKS, 128), jnp.int32),
        pltpu.VMEM((CHUNKS, 128), jnp.float32),
        pltpu.VMEM((RPS,), jnp.float32),
        pltpu.VMEM((128, H), jnp.float32),
        pltpu.VMEM((128, H), jnp.float32),
        pltpu.VMEM_SHARED((NP, H), jnp.float32),
        pltpu.SemaphoreType.DMA,
        pltpu.SemaphoreType.DMA,
    ],
)
def _hop_kernel(u2, srcr, dstr, ewr, dinv, out, srcb, dstb, nb, dvb, rA, rB,
                acc, semA, semB):
    c = lax.axis_index("c")
    s = lax.axis_index("s")
    pltpu.sync_copy(srcr.at[pl.ds(s * CHUNKS, CHUNKS)], srcb)
    pltpu.sync_copy(dstr.at[pl.ds(s * CHUNKS, CHUNKS)], dstb)
    pltpu.sync_copy(ewr.at[pl.ds(s * CHUNKS, CHUNKS)], nb)
    pltpu.sync_copy(dinv.at[pl.ds(s * RPS, RPS)], dvb)

    off = jnp.broadcast_to(c * NP, (16,)).astype(jnp.int32)

    def offbody(j, _):
        for v in range(8):
            sl = pl.ds(v * 16, 16)
            srcb[j, sl] = srcb[j, sl] + off
        return 0

    lax.fori_loop(0, CHUNKS, offbody, 0)

    zer = jnp.zeros((16,), jnp.float32)

    def zbody(r, _):
        for v in range(H // 16):
            rA[r, pl.ds(v * 16, 16)] = zer
        return 0

    lax.fori_loop(0, 128, zbody, 0)
    for k in range(RPS // 128):
        pltpu.sync_copy(rA, acc.at[pl.ds(s * RPS + k * 128, 128)])
    plsc.subcore_barrier()

    def scale_and_scatter(j, rbuf):
        def gbody(g, _):
            nv = nb[j, pl.ds(g * 16, 16)]
            for l in range(16):
                scv = jnp.broadcast_to(nv[l], (16,))
                r = g * 16 + l
                for v in range(H // 16):
                    sl = pl.ds(v * 16, 16)
                    rbuf[r, sl] = rbuf[r, sl] * scv
            return 0

        lax.fori_loop(0, 8, gbody, 0)
        pltpu.sync_copy(rbuf, acc.at[dstb.at[j]], add=True)

    pltpu.async_copy(u2.at[srcb.at[0]], rA, semA)

    def loop(jj, _):
        j0 = 2 * jj
        j1 = j0 + 1
        pltpu.make_async_copy(u2.at[srcb.at[j0]], rA, semA).wait()
        pltpu.async_copy(u2.at[srcb.at[j1]], rB, semB)
        scale_and_scatter(j0, rA)
        pltpu.make_async_copy(u2.at[srcb.at[j1]], rB, semB).wait()

        @pl.when(jj < CHUNKS // 2 - 1)
        def _():
            pltpu.async_copy(u2.at[srcb.at[j0 + 2]], rA, semA)

        scale_and_scatter(j1, rB)
        return 0

    lax.fori_loop(0, CHUNKS // 2, loop, 0)
    plsc.subcore_barrier()

    for k in range(RPS // 128):
        pltpu.sync_copy(acc.at[pl.ds(s * RPS + k * 128, 128)], rA)

        def wbody(g, _):
            dv = dvb[pl.ds(k * 128 + g * 16, 16)]
            for l in range(16):
                d = dv[l]
                d2 = jnp.broadcast_to(d * d, (16,))
                r = g * 16 + l
                for v in range(H // 16):
                    sl = pl.ds(v * 16, 16)
                    rA[r, sl] = rA[r, sl] * d2
            return 0

        lax.fori_loop(0, 8, wbody, 0)
        pltpu.sync_copy(rA, out.at[pl.ds(c * NP + s * RPS + k * 128, 128)])


BLK = 512


def _rowscale_body(x_ref, d_ref, out_ref):
    d = d_ref[...]
    out_ref[0] = x_ref[0] * d
    out_ref[1] = x_ref[1] * d


def _rowscale(x2, dinv_col):
    return pl.pallas_call(
        _rowscale_body,
        grid=(NP // BLK,),
        in_specs=[pl.BlockSpec((2, BLK, H), lambda r: (0, r, 0)),
                  pl.BlockSpec((BLK, 1), lambda r: (r, 0))],
        out_specs=pl.BlockSpec((2, BLK, H), lambda r: (0, r, 0)),
        out_shape=jax.ShapeDtypeStruct((2, NP, H), jnp.float32),
    )(x2, dinv_col)


def _mm_dots(h0, us, dsq, W_ref):
    d = dsq[...]
    res = jnp.zeros((BLK, D), jnp.float32)
    for i in range(4):
        if i == 0:
            a = h0[0]
            b = h0[1]
        else:
            a = us[i - 1][0] * d
            b = us[i - 1][1] * d
        wa = W_ref[pl.ds(i * D, H), :]
        wb = W_ref[pl.ds(i * D + H, H), :]
        res = res + jnp.dot(a, wa, preferred_element_type=jnp.float32,
                            precision=lax.Precision.HIGHEST)
        res = res + jnp.dot(b, wb, preferred_element_type=jnp.float32,
                            precision=lax.Precision.HIGHEST)
    return res


def _mm1_body(h0, u1, u2, u3, dsq, dinv, W_ref, b_ref, out_ref, us_ref):
    res = _mm_dots(h0, (u1, u2, u3), dsq, W_ref) + b_ref[...]
    res = jnp.where(res >= 0, res, 0.01 * res)
    out_ref[0] = res[:, :H]
    out_ref[1] = res[:, H:]
    dv = dinv[...]
    us_ref[0] = out_ref[0] * dv
    us_ref[1] = out_ref[1] * dv


def _mm2_body(h0, u1, u2, u3, dsq, W_ref, b_ref, m_ref, out_ref):
    res = _mm_dots(h0, (u1, u2, u3), dsq, W_ref) + b_ref[...]
    out_ref[...] = res * m_ref[...]


_h_spec = pl.BlockSpec((2, BLK, H), lambda r: (0, r, 0))
_c_spec = pl.BlockSpec((BLK, 1), lambda r: (r, 0))
_w_spec = pl.BlockSpec((4 * D, D), lambda r: (0, 0))
_b_spec = pl.BlockSpec((1, D), lambda r: (0, 0))


def _mm1(h0, u1, u2, u3, dsq, dinv_col, W, b):
    return pl.pallas_call(
        _mm1_body,
        grid=(NP // BLK,),
        in_specs=[_h_spec, _h_spec, _h_spec, _h_spec, _c_spec, _c_spec,
                  _w_spec, _b_spec],
        out_specs=(_h_spec, _h_spec),
        out_shape=(jax.ShapeDtypeStruct((2, NP, H), jnp.float32),
                   jax.ShapeDtypeStruct((2, NP, H), jnp.float32)),
    )(h0, u1, u2, u3, dsq, dinv_col, W, b)


def _mm2(h0, u1, u2, u3, dsq, W, b, m):
    return pl.pallas_call(
        _mm2_body,
        grid=(NP // BLK,),
        in_specs=[_h_spec, _h_spec, _h_spec, _h_spec, _c_spec, _w_spec,
                  _b_spec, _c_spec],
        out_specs=pl.BlockSpec((BLK, D), lambda r: (r, 0)),
        out_shape=jax.ShapeDtypeStruct((NP, D), jnp.float32),
    )(h0, u1, u2, u3, dsq, W, b, m)


def kernel(x, edge_index, edge_weights, feature_mask, W0, b0, W1, b1):
    src = edge_index[0]
    dst = edge_index[1]
    pad = EP - E
    padidx = (jnp.arange(pad, dtype=jnp.int32) * 97) % N
    srcr = jnp.concatenate([src, padidx]).reshape(EP // 128, 128)
    dstr = jnp.concatenate([dst, padidx]).reshape(EP // 128, 128)
    ewr = jnp.concatenate(
        [edge_weights, jnp.zeros((pad,), jnp.float32)]).reshape(EP // 128, 128)

    xp = jnp.zeros((NP, D), jnp.float32).at[:N].set(x)
    x2 = jnp.concatenate([xp[:, :H], xp[:, H:]], axis=0).reshape(2, NP, H)

    degp = _deg_kernel(dstr, ewr)
    dinv, dsq = _dinv_kernel(degp.reshape(2 * NP // 128, 128))
    dinv_flat = dinv.reshape(NP)
    dinv_col = dinv.reshape(NP, 1)
    dsq_col = dsq.reshape(NP, 1)

    u0 = _rowscale(x2, dinv_col).reshape(2 * NP, H)
    u1 = _hop_kernel(u0, srcr, dstr, ewr, dinv_flat)
    u2 = _hop_kernel(u1, srcr, dstr, ewr, dinv_flat)
    u3 = _hop_kernel(u2, srcr, dstr, ewr, dinv_flat)

    def f(a):
        return a.reshape(2, NP, H)

    out1, ub = _mm1(x2, f(u1), f(u2), f(u3), dsq_col, dinv_col, W0,
                    b0.reshape(1, D))

    ub = ub.reshape(2 * NP, H)
    v1 = _hop_kernel(ub, srcr, dstr, ewr, dinv_flat)
    v2 = _hop_kernel(v1, srcr, dstr, ewr, dinv_flat)
    v3 = _hop_kernel(v2, srcr, dstr, ewr, dinv_flat)

    maskp = jnp.zeros((NP, 1), jnp.float32).at[:N, 0].set(feature_mask)
    out2 = _mm2(out1, f(v1), f(v2), f(v3), dsq_col, W1, b1.reshape(1, D),
                maskp)
    return out2[:N]


# speedup vs baseline: 10.9010x; 1.0981x over previous
import functools

import jax
import jax.numpy as jnp
from jax import lax
from jax.experimental import pallas as pl
from jax.experimental.pallas import tpu as pltpu
from jax.experimental.pallas import tpu_sc as plsc

N = 10000
E = 320000
D = 128
H = D // 2
NP = 10240
NC = 2
NS = 16
RPS = NP // NS
ET = 20480
EP = NS * ET
CHUNKS = ET // 128
WCHUNKS = CHUNKS // 2

_mesh = plsc.VectorSubcoreMesh(core_axis_name="c", subcore_axis_name="s")


@functools.partial(
    pl.kernel,
    out_type=jax.ShapeDtypeStruct((NC, NP), jnp.float32),
    mesh=_mesh,
    compiler_params=pltpu.CompilerParams(use_tc_tiling_on_sc=False),
    scratch_types=[
        pltpu.VMEM((WCHUNKS, 128), jnp.int32),
        pltpu.VMEM((WCHUNKS, 128), jnp.float32),
        pltpu.VMEM((RPS,), jnp.float32),
        pltpu.VMEM_SHARED((NP,), jnp.float32),
    ],
)
def _deg_kernel(dstr, ewr, degp, dstb, ewb, zb, dacc):
    c = lax.axis_index("c")
    s = lax.axis_index("s")
    wid = s * NC + c
    pltpu.sync_copy(dstr.at[pl.ds(wid * WCHUNKS, WCHUNKS)], dstb)
    pltpu.sync_copy(ewr.at[pl.ds(wid * WCHUNKS, WCHUNKS)], ewb)
    zer = jnp.zeros((16,), jnp.float32)

    def zbody(i, _):
        zb[pl.ds(i * 16, 16)] = zer
        return 0

    lax.fori_loop(0, RPS // 16, zbody, 0)
    pltpu.sync_copy(zb, dacc.at[pl.ds(s * RPS, RPS)])
    plsc.subcore_barrier()

    def body(j, _):
        pltpu.sync_copy(ewb.at[j], dacc.at[dstb.at[j]], add=True)
        return 0

    lax.fori_loop(0, WCHUNKS, body, 0)
    plsc.subcore_barrier()
    pltpu.sync_copy(dacc.at[pl.ds(s * RPS, RPS)],
                    degp.at[c, pl.ds(s * RPS, RPS)])


def _dinv_body(degr_ref, dinv_ref, dsq_ref):
    d = degr_ref[0:NP // 128, :] + degr_ref[NP // 128:2 * (NP // 128), :]
    pos = d > 0
    dinv_ref[...] = jnp.where(pos, lax.rsqrt(d), 0.0)
    dsq_ref[...] = jnp.where(pos, jnp.sqrt(d), 0.0)


def _dinv_kernel(degr):
    return pl.pallas_call(
        _dinv_body,
        out_shape=(jax.ShapeDtypeStruct((NP // 128, 128), jnp.float32),
                   jax.ShapeDtypeStruct((NP // 128, 128), jnp.float32)),
    )(degr)


@functools.partial(
    pl.kernel,
    out_type=jax.ShapeDtypeStruct((2 * NP, H), jnp.float32),
    mesh=_mesh,
    compiler_params=pltpu.CompilerParams(use_tc_tiling_on_sc=False),
    scratch_types=[
        pltpu.VMEM((ET // 2,), jnp.int32),
        pltpu.VMEM((CHUNKS // 2, 128), jnp.int32),
        pltpu.VMEM((CHUNKS // 2, 128), jnp.float32),
        pltpu.VMEM((RPS,), jnp.float32),
        pltpu.VMEM((128, H), jnp.float32),
        pltpu.VMEM((128, H), jnp.float32),
        pltpu.VMEM((128, H), jnp.float32),
        pltpu.VMEM((128, H), jnp.float32),
        pltpu.VMEM_SHARED((NP, H), jnp.float32),
        pltpu.SemaphoreType.DMA,
        pltpu.SemaphoreType.DMA,
        pltpu.SemaphoreType.DMA,
        pltpu.SemaphoreType.DMA,
        pltpu.SemaphoreType.DMA,
        pltpu.SemaphoreType.DMA,
        pltpu.SemaphoreType.DMA,
        pltpu.SemaphoreType.DMA,
    ],
)
def _hop_kernel(u2, srcf, dstr, ewr, dinv, out, srcb, dstb, ewb, dvb,
                r0, r1, r2, r3, acc, g0, g1, g2, g3, s0, s1, s2, s3):
    c = lax.axis_index("c")
    s = lax.axis_index("s")
    pltpu.sync_copy(dinv.at[pl.ds(s * RPS, RPS)], dvb)

    zer = jnp.zeros((16,), jnp.float32)

    def zbody(r, _):
        for v in range(H // 16):
            r0[r, pl.ds(v * 16, 16)] = zer
        return 0

    lax.fori_loop(0, 128, zbody, 0)
    for kk in range(RPS // 128):
        pltpu.sync_copy(r0, acc.at[pl.ds(s * RPS + kk * 128, 128)])
    plsc.subcore_barrier()

    bufs = (r0, r1, r2, r3)
    gsem = (g0, g1, g2, g3)
    ssem = (s0, s1, s2, s3)
    HCH = CHUNKS // 2
    off = jnp.broadcast_to(c * NP, (16,)).astype(jnp.int32)

    def gath(q, m):
        return pltpu.make_async_copy(
            u2.at[srcb.at[pl.ds(q * 128, 128)]], bufs[m], gsem[m])

    def scat_start(q, m):
        pltpu.async_copy(bufs[m], acc.at[dstb.at[q]], ssem[m], add=True)

    def scat_wait(q, m):
        pltpu.make_async_copy(bufs[m], acc.at[dstb.at[q]], ssem[m]).wait()

    def scale(q, rbuf):
        def gbody(g, _):
            nv = ewb[q, pl.ds(g * 16, 16)]
            for l in range(16):
                scv = jnp.broadcast_to(nv[l], (16,))
                r = g * 16 + l
                for v in range(H // 16):
                    sl = pl.ds(v * 16, 16)
                    rbuf[r, sl] = rbuf[r, sl] * scv
            return 0

        lax.fori_loop(0, 8, gbody, 0)

    for p in range(2):
        pltpu.sync_copy(srcf.at[pl.ds(s * ET + p * (ET // 2), ET // 2)], srcb)
        pltpu.sync_copy(dstr.at[pl.ds(s * CHUNKS + p * HCH, HCH)], dstb)
        pltpu.sync_copy(ewr.at[pl.ds(s * CHUNKS + p * HCH, HCH)], ewb)

        def offbody(j, _):
            srcb[pl.ds(j * 16, 16)] = srcb[pl.ds(j * 16, 16)] + off
            return 0

        lax.fori_loop(0, (ET // 2) // 16, offbody, 0)

        gath(0, 0).start()
        gath(1, 1).start()

        def loop(i, _):
            for k in range(4):
                q = 4 * i + k
                m = (k + 2) % 4
                gath(q, k).wait()
                scale(q, bufs[k])
                scat_start(q, k)
                if k < 2:
                    @pl.when(i > 0)
                    def _():
                        scat_wait(q - 2, m)
                else:
                    scat_wait(q - 2, m)

                @pl.when(q + 2 < HCH)
                def _():
                    gath(q + 2, m).start()
            return 0

        lax.fori_loop(0, HCH // 4, loop, 0)
        scat_wait(HCH - 2, 2)
        scat_wait(HCH - 1, 3)

    plsc.subcore_barrier()

    for kk in range(RPS // 128):
        pltpu.sync_copy(acc.at[pl.ds(s * RPS + kk * 128, 128)], r0)

        def wbody(g, _):
            dv = dvb[pl.ds(kk * 128 + g * 16, 16)]
            for l in range(16):
                d = dv[l]
                d2 = jnp.broadcast_to(d * d, (16,))
                r = g * 16 + l
                for v in range(H // 16):
                    sl = pl.ds(v * 16, 16)
                    r0[r, sl] = r0[r, sl] * d2
            return 0

        lax.fori_loop(0, 8, wbody, 0)
        pltpu.sync_copy(r0, out.at[pl.ds(c * NP + s * RPS + kk * 128, 128)])


BLK = 512


def _rowscale_body(x_ref, d_ref, out_ref):
    d = d_ref[...]
    out_ref[0] = x_ref[0] * d
    out_ref[1] = x_ref[1] * d


def _rowscale(x2, dinv_col):
    return pl.pallas_call(
        _rowscale_body,
        grid=(NP // BLK,),
        in_specs=[pl.BlockSpec((2, BLK, H), lambda r: (0, r, 0)),
                  pl.BlockSpec((BLK, 1), lambda r: (r, 0))],
        out_specs=pl.BlockSpec((2, BLK, H), lambda r: (0, r, 0)),
        out_shape=jax.ShapeDtypeStruct((2, NP, H), jnp.float32),
    )(x2, dinv_col)


def _mm_dots(h0, us, dsq, W_ref):
    d = dsq[...]
    res = jnp.zeros((BLK, D), jnp.float32)
    for i in range(4):
        if i == 0:
            a = h0[0]
            b = h0[1]
        else:
            a = us[i - 1][0] * d
            b = us[i - 1][1] * d
        wa = W_ref[pl.ds(i * D, H), :]
        wb = W_ref[pl.ds(i * D + H, H), :]
        res = res + jnp.dot(a, wa, preferred_element_type=jnp.float32,
                            precision=lax.Precision.HIGHEST)
        res = res + jnp.dot(b, wb, preferred_element_type=jnp.float32,
                            precision=lax.Precision.HIGHEST)
    return res


def _mm1_body(h0, u1, u2, u3, dsq, dinv, W_ref, b_ref, out_ref, us_ref):
    res = _mm_dots(h0, (u1, u2, u3), dsq, W_ref) + b_ref[...]
    res = jnp.where(res >= 0, res, 0.01 * res)
    out_ref[0] = res[:, :H]
    out_ref[1] = res[:, H:]
    dv = dinv[...]
    us_ref[0] = out_ref[0] * dv
    us_ref[1] = out_ref[1] * dv


def _mm2_body(h0, u1, u2, u3, dsq, W_ref, b_ref, m_ref, out_ref):
    res = _mm_dots(h0, (u1, u2, u3), dsq, W_ref) + b_ref[...]
    out_ref[...] = res * m_ref[...]


_h_spec = pl.BlockSpec((2, BLK, H), lambda r: (0, r, 0))
_c_spec = pl.BlockSpec((BLK, 1), lambda r: (r, 0))
_w_spec = pl.BlockSpec((4 * D, D), lambda r: (0, 0))
_b_spec = pl.BlockSpec((1, D), lambda r: (0, 0))


def _mm1(h0, u1, u2, u3, dsq, dinv_col, W, b):
    return pl.pallas_call(
        _mm1_body,
        grid=(NP // BLK,),
        in_specs=[_h_spec, _h_spec, _h_spec, _h_spec, _c_spec, _c_spec,
                  _w_spec, _b_spec],
        out_specs=(_h_spec, _h_spec),
        out_shape=(jax.ShapeDtypeStruct((2, NP, H), jnp.float32),
                   jax.ShapeDtypeStruct((2, NP, H), jnp.float32)),
    )(h0, u1, u2, u3, dsq, dinv_col, W, b)


def _mm2(h0, u1, u2, u3, dsq, W, b, m):
    return pl.pallas_call(
        _mm2_body,
        grid=(NP // BLK,),
        in_specs=[_h_spec, _h_spec, _h_spec, _h_spec, _c_spec, _w_spec,
                  _b_spec, _c_spec],
        out_specs=pl.BlockSpec((BLK, D), lambda r: (r, 0)),
        out_shape=jax.ShapeDtypeStruct((NP, D), jnp.float32),
    )(h0, u1, u2, u3, dsq, W, b, m)


def kernel(x, edge_index, edge_weights, feature_mask, W0, b0, W1, b1):
    src = edge_index[0]
    dst = edge_index[1]
    pad = EP - E
    padidx = (jnp.arange(pad, dtype=jnp.int32) * 97) % N
    srcf = jnp.concatenate([src, padidx])
    dstr = jnp.concatenate([dst, padidx]).reshape(EP // 128, 128)
    ewr = jnp.concatenate(
        [edge_weights, jnp.zeros((pad,), jnp.float32)]).reshape(EP // 128, 128)

    xp = jnp.zeros((NP, D), jnp.float32).at[:N].set(x)
    x2 = jnp.concatenate([xp[:, :H], xp[:, H:]], axis=0).reshape(2, NP, H)

    degp = _deg_kernel(dstr, ewr)
    dinv, dsq = _dinv_kernel(degp.reshape(2 * NP // 128, 128))
    dinv_flat = dinv.reshape(NP)
    dinv_col = dinv.reshape(NP, 1)
    dsq_col = dsq.reshape(NP, 1)

    u0 = _rowscale(x2, dinv_col).reshape(2 * NP, H)
    u1 = _hop_kernel(u0, srcf, dstr, ewr, dinv_flat)
    u2 = _hop_kernel(u1, srcf, dstr, ewr, dinv_flat)
    u3 = _hop_kernel(u2, srcf, dstr, ewr, dinv_flat)

    def f(a):
        return a.reshape(2, NP, H)

    out1, ub = _mm1(x2, f(u1), f(u2), f(u3), dsq_col, dinv_col, W0,
                    b0.reshape(1, D))

    ub = ub.reshape(2 * NP, H)
    v1 = _hop_kernel(ub, srcf, dstr, ewr, dinv_flat)
    v2 = _hop_kernel(v1, srcf, dstr, ewr, dinv_flat)
    v3 = _hop_kernel(v2, srcf, dstr, ewr, dinv_flat)

    maskp = jnp.zeros((NP, 1), jnp.float32).at[:N, 0].set(feature_mask)
    out2 = _mm2(out1, f(v1), f(v2), f(v3), dsq_col, W1, b1.reshape(1, D),
                maskp)
    return out2[:N]
